# baseline (speedup 1.0000x reference)
import functools

import jax
import jax.numpy as jnp
from jax.experimental import pallas as pl

K = 4
VB = 16


def _tc_body(text_ref, vidT_ref, pooled_ref, idx_ref):
    text = text_ref[...]
    vidT = vidT_ref[...]
    F, vb, D = vidT.shape
    T = text.shape[0]

    sims = jax.lax.dot_general(
        vidT.reshape(F * vb, D), text,
        (((1,), (1,)), ((), ())),
        preferred_element_type=jnp.float32,
    ).reshape(F, vb, T)

    f_iota = jax.lax.broadcasted_iota(jnp.int32, (F, vb, T), 0)
    cur = sims
    idxs = []
    for j in range(K):
        m = jnp.max(cur, axis=0, keepdims=True)
        idx_j = jnp.min(jnp.where(cur == m, f_iota, F), axis=0)
        idxs.append(idx_j)
        cur = jnp.where(f_iota == idx_j[None], -jnp.inf, cur)

    idx = jnp.stack(idxs, axis=1)
    idx_ref[...] = jnp.broadcast_to(idx[:, :, None, :], (vb, K, D, T))

    oh = (cur == -jnp.inf).astype(jnp.float32)
    for v in range(vb):
        pooled_ref[v] = jax.lax.dot_general(
            oh[:, v, :], vidT[:, v, :], (((0,), (0,)), ((), ())),
            preferred_element_type=jnp.float32,
        )


@jax.jit
def kernel(text_embeds, video_embeds):
    T, D = text_embeds.shape
    V, F, _ = video_embeds.shape
    grid = (V // VB,)
    pooled, idx_exp = pl.pallas_call(
        _tc_body,
        grid=grid,
        in_specs=[
            pl.BlockSpec((T, D), lambda i: (0, 0)),
            pl.BlockSpec((F, VB, D), lambda i: (0, i, 0)),
        ],
        out_specs=[
            pl.BlockSpec((VB, T, D), lambda i: (i, 0, 0)),
            pl.BlockSpec((VB, K, D, T), lambda i: (i, 0, 0, 0)),
        ],
        out_shape=[
            jax.ShapeDtypeStruct((V, T, D), jnp.float32),
            jax.ShapeDtypeStruct((V, K, D, T), jnp.int32),
        ],
    )(text_embeds, jnp.transpose(video_embeds, (1, 0, 2)))
    return pooled, idx_exp


# speedup vs baseline: 1.0247x; 1.0247x over previous
import functools

import jax
import jax.numpy as jnp
from jax.experimental import pallas as pl

K = 4
VB = 8


def _tc_body(text_ref, vidT_ref, pooled_ref, idx_ref):
    text = text_ref[...]
    vidT = vidT_ref[...]
    F, vb, D = vidT.shape
    T = text.shape[0]

    sims = jax.lax.dot_general(
        vidT.reshape(F * vb, D), text,
        (((1,), (1,)), ((), ())),
        preferred_element_type=jnp.float32,
    ).reshape(F, vb, T)

    f_iota = jax.lax.broadcasted_iota(jnp.int32, (F, vb, T), 0)
    cur = sims
    idxs = []
    for j in range(K):
        m = jnp.max(cur, axis=0, keepdims=True)
        idx_j = jnp.min(jnp.where(cur == m, f_iota, F), axis=0)
        idxs.append(idx_j)
        cur = jnp.where(f_iota == idx_j[None], -jnp.inf, cur)

    idx = jnp.stack(idxs, axis=1)
    idx_ref[...] = jnp.broadcast_to(idx[:, :, None, :], (vb, K, D, T))

    oh = (cur == -jnp.inf).astype(jnp.float32)
    for v in range(vb):
        pooled_ref[v] = jax.lax.dot_general(
            oh[:, v, :], vidT[:, v, :], (((0,), (0,)), ((), ())),
            preferred_element_type=jnp.float32,
        )


@jax.jit
def kernel(text_embeds, video_embeds):
    T, D = text_embeds.shape
    V, F, _ = video_embeds.shape
    grid = (V // VB,)
    pooled, idx_exp = pl.pallas_call(
        _tc_body,
        grid=grid,
        in_specs=[
            pl.BlockSpec((T, D), lambda i: (0, 0)),
            pl.BlockSpec((F, VB, D), lambda i: (0, i, 0)),
        ],
        out_specs=[
            pl.BlockSpec((VB, T, D), lambda i: (i, 0, 0)),
            pl.BlockSpec((VB, K, D, T), lambda i: (i, 0, 0, 0)),
        ],
        out_shape=[
            jax.ShapeDtypeStruct((V, T, D), jnp.float32),
            jax.ShapeDtypeStruct((V, K, D, T), jnp.int32),
        ],
    )(text_embeds, jnp.transpose(video_embeds, (1, 0, 2)))
    return pooled, idx_exp


# speedup vs baseline: 1.0268x; 1.0021x over previous
import jax
import jax.numpy as jnp
from jax.experimental import pallas as pl

K = 4
VB = 8


def _tc_body(text_ref, vidT_ref, pooled_ref, idx_ref):
    text = text_ref[...]
    vidT = vidT_ref[...]
    F, vb, D = vidT.shape
    T = text.shape[0]

    sims = jax.lax.dot_general(
        vidT.reshape(F * vb, D), text,
        (((1,), (1,)), ((), ())),
        preferred_element_type=jnp.float32,
    ).reshape(F, vb, T)

    f_iota = jax.lax.broadcasted_iota(jnp.int32, (F, vb, T), 0)
    cur = sims
    idxs = []
    for j in range(K):
        m = jnp.max(cur, axis=0, keepdims=True)
        idx_j = jnp.min(jnp.where(cur == m, f_iota, F), axis=0)
        idxs.append(idx_j)
        cur = jnp.where(f_iota == idx_j[None], -jnp.inf, cur)

    idx = jnp.stack(idxs, axis=1)
    idx_ref[...] = jnp.broadcast_to(idx[:, :, None, :], (vb, K, D, T))

    oh = (cur == -jnp.inf).astype(jnp.float32)
    for v in range(vb):
        pooled_ref[v] = jax.lax.dot_general(
            oh[:, v, :], vidT[:, v, :], (((0,), (0,)), ((), ())),
            preferred_element_type=jnp.float32,
        )


@jax.jit
def kernel(text_embeds, video_embeds):
    T, D = text_embeds.shape
    V, F, _ = video_embeds.shape
    grid = (V // VB,)
    pooled, idx_exp = pl.pallas_call(
        _tc_body,
        grid=grid,
        in_specs=[
            pl.BlockSpec((T, D), lambda i: (0, 0)),
            pl.BlockSpec((F, VB, D), lambda i: (0, i, 0)),
        ],
        out_specs=[
            pl.BlockSpec((VB, T, D), lambda i: (i, 0, 0)),
            pl.BlockSpec((VB, K, D, T), lambda i: (i, 0, 0, 0)),
        ],
        out_shape=[
            jax.ShapeDtypeStruct((V, T, D), jnp.float32),
            jax.ShapeDtypeStruct((V, K, D, T), jnp.int32),
        ],
    )(text_embeds, jnp.transpose(video_embeds, (1, 0, 2)))
    return pooled, idx_exp
